# baseline (speedup 1.0000x reference)
import jax
import jax.numpy as jnp
from jax import lax
from jax.experimental import pallas as pl
from jax.experimental.pallas import tpu as pltpu
from jax.experimental.pallas import tpu_sc as plsc

NUM_CORES = 2
NUM_SUBCORES = 16
LANES = 16
NW = NUM_CORES * NUM_SUBCORES

IDX_CHUNK = 128


def _make_sc_kernel(B, D, E, S):
    assert B % NW == 0 and D == 64 and E % 2 == 0 and S % 2 == 0
    bpw = B // NW
    n_chunks = bpw // IDX_CHUNK
    grp_per_chunk = IDX_CHUNK // LANES
    mesh = plsc.VectorSubcoreMesh(core_axis_name="c", subcore_axis_name="s")

    def body(pairs_hbm, ew_hbm, sw_hbm, out_hbm,
             pairs_v, eidx_v, ecol_v, sidx_v, erows_v, stab_v, out_v,
             sem, stab_sem):
        wid = lax.axis_index("s") * NUM_CORES + lax.axis_index("c")
        base = wid * bpw
        lane = lax.iota(jnp.int32, LANES)

        stab_cp = pltpu.async_copy(sw_hbm, stab_v, stab_sem)

        pltpu.sync_copy(pairs_hbm.at[pl.ds(base * 2, bpw * 2)], pairs_v)

        for c in range(bpw // LANES):
            src = c * 2 * LANES + lane * 2
            ei = plsc.load_gather(pairs_v, [src])
            si = plsc.load_gather(pairs_v, [src + 1])
            ei = jnp.minimum(jnp.maximum(ei, 0), E - 1)
            si = jnp.minimum(jnp.maximum(si, 0), S - 1)
            row, off = divmod(c * LANES, IDX_CHUNK)
            eidx_v[row, pl.ds(off, LANES)] = ei >> 1
            ecol_v[pl.ds(c * LANES, LANES)] = (ei & 1) * D
            sidx_v[pl.ds(c * LANES, LANES)] = si

        def start(j):
            buf = erows_v.at[j % 2]
            return pltpu.async_copy(ew_hbm.at[eidx_v.at[j]], buf, sem)

        cps = [start(0), start(1)]
        stab_cp.wait()

        for j in range(n_chunks):
            cps[j % 2].wait()
            for g in range(grp_per_chunk):
                i0 = j * IDX_CHUNK + g * LANES
                r = g * LANES + lane
                ecol = ecol_v[pl.ds(i0, LANES)]
                sid = sidx_v[pl.ds(i0, LANES)]
                srow = sid >> 1
                scol = (sid & 1) * D
                acc = jnp.zeros((LANES,), jnp.float32)
                for d in range(D):
                    ev = plsc.load_gather(erows_v, [jnp.full((LANES,), j % 2, jnp.int32), r, ecol + d])
                    sv = plsc.load_gather(stab_v, [srow, scol + d])
                    acc = acc + ev * sv
                out_v[pl.ds(i0, LANES)] = 1.0 / (1.0 + jnp.exp(-acc))
            if j + 2 < n_chunks:
                cps[j % 2] = start(j + 2)

        pltpu.sync_copy(out_v, out_hbm.at[pl.ds(base, bpw)])

    return pl.kernel(
        body,
        out_type=jax.ShapeDtypeStruct((B,), jnp.float32),
        mesh=mesh,
        compiler_params=pltpu.CompilerParams(
            needs_layout_passes=False, use_tc_tiling_on_sc=True),
        scratch_types=[
            pltpu.VMEM((2 * bpw,), jnp.int32),
            pltpu.VMEM((n_chunks, IDX_CHUNK), jnp.int32),
            pltpu.VMEM((bpw,), jnp.int32),
            pltpu.VMEM((bpw,), jnp.int32),
            pltpu.VMEM((2, IDX_CHUNK, 2 * D), jnp.float32),
            pltpu.VMEM((S // 2, 2 * D), jnp.float32),
            pltpu.VMEM((bpw,), jnp.float32),
            pltpu.SemaphoreType.DMA,
            pltpu.SemaphoreType.DMA,
        ],
    )


def kernel(pairs, exercise_w, skill_w):
    B = pairs.shape[0]
    E, D = exercise_w.shape
    S = skill_w.shape[0]
    sc = _make_sc_kernel(B, D, E, S)
    return sc(pairs.reshape(-1),
              exercise_w.reshape(E // 2, 2 * D),
              skill_w.reshape(S // 2, 2 * D))


# speedup vs baseline: 1.0534x; 1.0534x over previous
import jax
import jax.numpy as jnp
from jax import lax
from jax.experimental import pallas as pl
from jax.experimental.pallas import tpu as pltpu
from jax.experimental.pallas import tpu_sc as plsc

NUM_CORES = 2
NUM_SUBCORES = 16
LANES = 16
NW = NUM_CORES * NUM_SUBCORES

ROWS_PER_CHUNK = 2
N_PASSES = 2


def _make_sc_kernel(B, D, E, S):
    assert B % NW == 0 and D % (N_PASSES * ROWS_PER_CHUNK) == 0
    bpw = B // NW
    d_half = D // N_PASSES
    cpp = d_half // ROWS_PER_CHUNK
    n_chunks = N_PASSES * cpp
    n_groups = bpw // LANES
    mesh = plsc.VectorSubcoreMesh(core_axis_name="c", subcore_axis_name="s")

    def body(pairs_hbm, ewt_hbm, swt_hbm, out_hbm,
             eids_v, sids_v, swt_v, ev_v, acc_v, out_v,
             spbufs, sem_sp0, sem_sp1, sem_ev, sem_sw):
        sid = lax.axis_index("s")
        wid = sid * NUM_CORES + lax.axis_index("c")
        base = wid * bpw
        sems = [sem_sp0, sem_sp1]

        sw_cp = pltpu.async_copy(swt_hbm, swt_v, sem_sw)

        pltpu.sync_copy(pairs_hbm.at[0, pl.ds(base, bpw)], eids_v)
        pltpu.sync_copy(pairs_hbm.at[1, pl.ds(base, bpw)], sids_v)
        for c in range(n_groups):
            sl = pl.ds(c * LANES, LANES)
            eids_v[sl] = jnp.minimum(jnp.maximum(eids_v[sl], 0), E - 1)
            sids_v[sl] = jnp.minimum(jnp.maximum(sids_v[sl], 0), S - 1)

        def stage(c):
            pltpu.async_copy(
                ewt_hbm.at[pl.ds(c * ROWS_PER_CHUNK, ROWS_PER_CHUNK)],
                spbufs.at[c % 2], sems[c % 2])

        def wait_stage(c):
            pltpu.make_async_copy(
                ewt_hbm.at[pl.ds(c * ROWS_PER_CHUNK, ROWS_PER_CHUNK)],
                spbufs.at[c % 2], sems[c % 2]).wait()

        @pl.when(sid == 0)
        def _():
            stage(0)

        sw_cp.wait()

        for h in range(N_PASSES):
            for cc in range(cpp):
                c = h * cpp + cc

                @pl.when(sid == 0)
                def _(c=c):
                    wait_stage(c)

                plsc.subcore_barrier()
                if c + 1 < n_chunks:
                    @pl.when(sid == 0)
                    def _(c=c):
                        stage(c + 1)

                ev_cps = []
                for j in range(ROWS_PER_CHUNK):
                    ld = cc * ROWS_PER_CHUNK + j
                    ev_cps.append(pltpu.async_copy(
                        spbufs.at[c % 2].at[j].at[eids_v],
                        ev_v.at[ld], sem_ev))
                for cp in ev_cps:
                    cp.wait()

            def g_body(g, carry, h=h):
                i0 = g * LANES
                sid_vec = sids_v[pl.ds(i0, LANES)]
                acc = jnp.zeros((LANES,), jnp.float32)
                for ld in range(d_half):
                    ev = ev_v[ld, pl.ds(i0, LANES)]
                    sv = plsc.load_gather(
                        swt_v, [jnp.full((LANES,), h * d_half + ld, jnp.int32), sid_vec])
                    acc = acc + ev * sv
                if h == 0:
                    acc_v[pl.ds(i0, LANES)] = acc
                else:
                    total = acc_v[pl.ds(i0, LANES)] + acc
                    out_v[pl.ds(i0, LANES)] = 1.0 / (1.0 + jnp.exp(-total))
                return carry

            lax.fori_loop(0, n_groups, g_body, 0)

        pltpu.sync_copy(out_v, out_hbm.at[pl.ds(base, bpw)])

    return pl.kernel(
        body,
        out_type=jax.ShapeDtypeStruct((B,), jnp.float32),
        mesh=mesh,
        compiler_params=pltpu.CompilerParams(
            needs_layout_passes=False, use_tc_tiling_on_sc=False),
        scratch_types=[
            pltpu.VMEM((bpw,), jnp.int32),
            pltpu.VMEM((bpw,), jnp.int32),
            pltpu.VMEM((D, S), jnp.float32),
            pltpu.VMEM((D // N_PASSES, bpw), jnp.float32),
            pltpu.VMEM((bpw,), jnp.float32),
            pltpu.VMEM((bpw,), jnp.float32),
            pltpu.VMEM_SHARED((2, ROWS_PER_CHUNK, E), jnp.float32),
            pltpu.SemaphoreType.DMA,
            pltpu.SemaphoreType.DMA,
            pltpu.SemaphoreType.DMA,
            pltpu.SemaphoreType.DMA,
        ],
    )


def kernel(pairs, exercise_w, skill_w):
    B = pairs.shape[0]
    E, D = exercise_w.shape
    S = skill_w.shape[0]
    sc = _make_sc_kernel(B, D, E, S)
    return sc(pairs.T, exercise_w.T, skill_w.T)
